# baseline (speedup 1.0000x reference)
import functools

import jax
import jax.numpy as jnp
from jax import lax
from jax.experimental import pallas as pl
from jax.experimental.pallas import tpu as pltpu
from jax.experimental.pallas import tpu_sc as plsc

N_NODES = 10000
D_EMB = 64
N_LAYERS = 4
N_EDGES_IN = 320000
E_TOT = N_EDGES_IN + N_NODES
NPAD = 10240
E_PAD = 331776
NW = 32
CHUNK = E_PAD // NW
BLK = 288
NBLK = CHUNK // BLK
NPAIR = NBLK // 2
GPB = BLK // 16
SL = NPAD // 16

_mesh = plsc.VectorSubcoreMesh(core_axis_name="c", subcore_axis_name="s")
_sc_params = pltpu.CompilerParams(needs_layout_passes=False,
                                  use_tc_tiling_on_sc=False)


def _leaky(v, s):
    return jnp.maximum(v, 0.0) + s * jnp.minimum(v, 0.0)


def _tc_front_body(x_ref, gm_ref, bt_ref, mu_ref, vr_ref, wi_ref, bi_ref,
                   w0_ref, b0_ref, xl_ref):
    x = x_ref[...]
    inv = lax.rsqrt(vr_ref[...] + 1e-5)
    h = (x - mu_ref[...]) * inv * gm_ref[...] + bt_ref[...]
    h = _leaky(jnp.dot(h, wi_ref[...], preferred_element_type=jnp.float32)
               + bi_ref[...], 0.01)
    xl = jnp.dot(h, w0_ref[...], preferred_element_type=jnp.float32) + b0_ref[...]
    xl_ref[...] = jnp.zeros((NPAD, D_EMB), jnp.float32)
    xl_ref[0:N_NODES, :] = xl


def _tc_front(x, gamma, beta, run_mean, run_var, W_in, b_in, W0, b0):
    return pl.pallas_call(
        _tc_front_body,
        out_shape=jax.ShapeDtypeStruct((NPAD, D_EMB), jnp.float32),
    )(x, gamma.reshape(1, -1), beta.reshape(1, -1), run_mean.reshape(1, -1),
      run_var.reshape(1, -1), W_in, b_in.reshape(1, -1), W0, b0.reshape(1, -1))


def _graphnorm(aggu_ref, den_ref, cb_ref, gw_ref, gb_ref, gms_ref):
    a = aggu_ref[0] + aggu_ref[1]
    den_all = den_ref[0] + den_ref[1]
    agg = (a[0:N_NODES, :] / (den_all[0:N_NODES, :] + 1e-16)) + cb_ref[...]
    mean = jnp.mean(agg, axis=0, keepdims=True)
    out = agg - mean * gms_ref[...]
    var = jnp.mean(out * out, axis=0, keepdims=True)
    h = gw_ref[...] * out * lax.rsqrt(var + 1e-5) + gb_ref[...]
    return _leaky(h, 0.01), den_all


def _tc_mid_body(aggu_ref, den_ref, cb_ref, gw_ref, gb_ref, gms_ref, w_ref,
                 b_ref, xl_ref, dout_ref):
    h, den_all = _graphnorm(aggu_ref, den_ref, cb_ref, gw_ref, gb_ref, gms_ref)
    xl = jnp.dot(h, w_ref[...], preferred_element_type=jnp.float32) + b_ref[...]
    xl_ref[...] = jnp.zeros((NPAD, D_EMB), jnp.float32)
    xl_ref[0:N_NODES, :] = xl
    dout_ref[...] = den_all


def _tc_mid(aggu, den, conv_b, gn_w, gn_b, gn_ms, W, b):
    return pl.pallas_call(
        _tc_mid_body,
        out_shape=(jax.ShapeDtypeStruct((NPAD, D_EMB), jnp.float32),
                   jax.ShapeDtypeStruct((NPAD, 1), jnp.float32)),
    )(aggu, den, conv_b.reshape(1, -1), gn_w.reshape(1, -1),
      gn_b.reshape(1, -1), gn_ms.reshape(1, -1), W, b.reshape(1, -1))


def _tc_last_body(aggu_ref, den_ref, cb_ref, gw_ref, gb_ref, gms_ref, h_ref,
                  dout_ref):
    h, den_all = _graphnorm(aggu_ref, den_ref, cb_ref, gw_ref, gb_ref, gms_ref)
    h_ref[...] = h
    dout_ref[...] = den_all


def _tc_last(aggu, den, conv_b, gn_w, gn_b, gn_ms):
    return pl.pallas_call(
        _tc_last_body,
        out_shape=(jax.ShapeDtypeStruct((N_NODES, D_EMB), jnp.float32),
                   jax.ShapeDtypeStruct((NPAD, 1), jnp.float32)),
    )(aggu, den, conv_b.reshape(1, -1), gn_w.reshape(1, -1),
      gn_b.reshape(1, -1), gn_ms.reshape(1, -1))


def _k1_body(xl_hbm, src_hbm, dst_hbm, att_hbm, e_out, m_out,
             idx0, idx1, rows0, rows1, e0, e1, m_loc, att_v, m_ca, m_cb,
             m_sh, sem0, sem1):
    c = lax.axis_index("c")
    s = lax.axis_index("s")
    wid = s * 2 + c
    base = pl.multiple_of(wid * CHUNK, 8)
    pltpu.sync_copy(att_hbm, att_v)

    neg = jnp.full((16,), -3e38, jnp.float32)

    def init_b(i, _):
        m_loc[pl.ds(i * 16, 16)] = neg
        return 0
    lax.fori_loop(0, NPAD // 16, init_b, 0)

    att_c = [att_v[pl.ds(k * 16, 16)] for k in range(D_EMB // 16)]

    def issue(b, idx, rows, sem):
        off = pl.multiple_of(base + b * BLK, 8)
        pltpu.sync_copy(src_hbm.at[pl.ds(off, BLK)], idx.at[pl.ds(0, BLK)])
        pltpu.sync_copy(dst_hbm.at[pl.ds(off, BLK)], idx.at[pl.ds(BLK, BLK)])
        pltpu.async_copy(xl_hbm.at[idx], rows, sem)

    def wait(idx, rows, sem):
        pltpu.make_async_copy(xl_hbm.at[idx], rows, sem).wait()

    def compute(b, idx, rows, ebuf):
        def grp(g, _):
            ids = g * 16 + lax.iota(jnp.int32, 16)
            acc = jnp.zeros((16,), jnp.float32)
            for d in range(D_EMB):
                dd = jnp.full((16,), d, jnp.int32)
                v = (plsc.load_gather(rows, [ids, dd])
                     + plsc.load_gather(rows, [BLK + ids, dd]))
                acc = acc + att_c[d // 16][d % 16] * _leaky(v, 0.2)
            ebuf[pl.ds(g * 16, 16)] = acc

            dst_v = idx[pl.ds(BLK + g * 16, 16)]

            def upd_cond(pending):
                return jnp.any(pending)

            def upd_body(pending):
                cur = plsc.load_gather(m_loc, [dst_v])
                plsc.store_scatter(m_loc, [dst_v],
                                   jnp.maximum(cur, acc), mask=pending)
                chk = plsc.load_gather(m_loc, [dst_v])
                return pending & (chk < acc)
            lax.while_loop(upd_cond, upd_body, jnp.ones((16,), jnp.bool_))
            return 0
        lax.fori_loop(0, GPB, grp, 0)
        off = pl.multiple_of(base + b * BLK, 8)
        pltpu.sync_copy(ebuf, e_out.at[pl.ds(off, BLK)])

    issue(0, idx0, rows0, sem0)

    def pair(h, _):
        b0 = 2 * h
        issue(b0 + 1, idx1, rows1, sem1)
        wait(idx0, rows0, sem0)
        compute(b0, idx0, rows0, e0)

        @pl.when(h + 1 < NPAIR)
        def _():
            issue(b0 + 2, idx0, rows0, sem0)
        wait(idx1, rows1, sem1)
        compute(b0 + 1, idx1, rows1, e1)
        return 0
    lax.fori_loop(0, NPAIR, pair, 0)

    pltpu.sync_copy(m_loc, m_sh.at[s])
    plsc.subcore_barrier()
    colo = pl.multiple_of(s * SL, 8)
    pltpu.sync_copy(m_sh.at[0, pl.ds(colo, SL)], m_ca)

    def red_j(j, _):
        pltpu.sync_copy(m_sh.at[j, pl.ds(colo, SL)], m_cb)

        def mx(k, _):
            sl = pl.ds(k * 16, 16)
            m_ca[sl] = jnp.maximum(m_ca[sl], m_cb[sl])
            return 0
        lax.fori_loop(0, SL // 16, mx, 0)
        return 0
    lax.fori_loop(1, 16, red_j, 0)
    pltpu.sync_copy(m_ca, m_out.at[c, pl.ds(colo, SL)])


@functools.partial(
    pl.kernel, mesh=_mesh, compiler_params=_sc_params,
    out_type=(jax.ShapeDtypeStruct((E_PAD,), jnp.float32),
              jax.ShapeDtypeStruct((2, NPAD), jnp.float32)),
    scratch_types=[
        pltpu.VMEM((2 * BLK,), jnp.int32),
        pltpu.VMEM((2 * BLK,), jnp.int32),
        pltpu.VMEM((2 * BLK, D_EMB), jnp.float32),
        pltpu.VMEM((2 * BLK, D_EMB), jnp.float32),
        pltpu.VMEM((BLK,), jnp.float32),
        pltpu.VMEM((BLK,), jnp.float32),
        pltpu.VMEM((NPAD,), jnp.float32),
        pltpu.VMEM((D_EMB,), jnp.float32),
        pltpu.VMEM((SL,), jnp.float32),
        pltpu.VMEM((SL,), jnp.float32),
        pltpu.VMEM_SHARED((16, NPAD), jnp.float32),
        pltpu.SemaphoreType.DMA,
        pltpu.SemaphoreType.DMA,
    ])
def _k1(xl_hbm, src_hbm, dst_hbm, att_hbm, e_out, m_out, *scratch):
    _k1_body(xl_hbm, src_hbm, dst_hbm, att_hbm, e_out, m_out, *scratch)


def _k2_body(xl_hbm, src_hbm, dst_hbm, e_hbm, m_hbm, ee_out, agg_out, den_out,
             si0, si1, di0, di1, rows0, rows1, e0, e1, m_loc, m_tmp,
             zbuf, zvec, agg_sh, den_sh, sem0, sem1):
    c = lax.axis_index("c")
    s = lax.axis_index("s")
    wid = s * 2 + c
    base = pl.multiple_of(wid * CHUNK, 8)

    pltpu.sync_copy(m_hbm.at[0], m_loc)
    pltpu.sync_copy(m_hbm.at[1], m_tmp)

    def mx(k, _):
        sl = pl.ds(k * 16, 16)
        m_loc[sl] = jnp.maximum(m_loc[sl], m_tmp[sl])
        return 0
    lax.fori_loop(0, NPAD // 16, mx, 0)

    zero16 = jnp.zeros((16,), jnp.float32)

    def zr(r, _):
        def zc(k, _):
            zbuf[r, pl.ds(k * 16, 16)] = zero16
            return 0
        lax.fori_loop(0, D_EMB // 16, zc, 0)
        return 0
    lax.fori_loop(0, SL // 2, zr, 0)

    def zv(k, _):
        zvec[pl.ds(k * 16, 16)] = zero16
        return 0
    lax.fori_loop(0, SL // 16, zv, 0)
    rowo = pl.multiple_of(s * SL, 8)
    pltpu.sync_copy(zbuf, agg_sh.at[pl.ds(rowo, SL // 2), :])
    pltpu.sync_copy(zbuf, agg_sh.at[pl.ds(rowo + SL // 2, SL // 2), :])
    pltpu.sync_copy(zvec, den_sh.at[pl.ds(rowo, SL)])
    plsc.subcore_barrier()

    def issue(b, si, di, ebuf, rows, sem):
        off = pl.multiple_of(base + b * BLK, 8)
        pltpu.sync_copy(src_hbm.at[pl.ds(off, BLK)], si)
        pltpu.sync_copy(dst_hbm.at[pl.ds(off, BLK)], di)
        pltpu.sync_copy(e_hbm.at[pl.ds(off, BLK)], ebuf)
        pltpu.async_copy(xl_hbm.at[si], rows, sem)

    def wait(si, rows, sem):
        pltpu.make_async_copy(xl_hbm.at[si], rows, sem).wait()

    def compute(b, di, ebuf, rows):
        def grp(g, _):
            sl = pl.ds(g * 16, 16)
            ids = g * 16 + lax.iota(jnp.int32, 16)
            dst_v = di[sl]
            mg = plsc.load_gather(m_loc, [dst_v])
            ee = jnp.exp(ebuf[sl] - mg)
            ebuf[sl] = ee

            for d in range(D_EMB):
                dd = jnp.full((16,), d, jnp.int32)
                col = plsc.load_gather(rows, [ids, dd])
                plsc.store_scatter(rows, [ids, dd], col * ee)
            return 0
        lax.fori_loop(0, GPB, grp, 0)
        pltpu.sync_copy(rows, agg_sh.at[di], add=True)
        pltpu.sync_copy(ebuf, den_sh.at[di], add=True)
        off = pl.multiple_of(base + b * BLK, 8)
        pltpu.sync_copy(ebuf, ee_out.at[pl.ds(off, BLK)])

    issue(0, si0, di0, e0, rows0, sem0)

    def pair(h, _):
        b0 = 2 * h
        issue(b0 + 1, si1, di1, e1, rows1, sem1)
        wait(si0, rows0, sem0)
        compute(b0, di0, e0, rows0)

        @pl.when(h + 1 < NPAIR)
        def _():
            issue(b0 + 2, si0, di0, e0, rows0, sem0)
        wait(si1, rows1, sem1)
        compute(b0 + 1, di1, e1, rows1)
        return 0
    lax.fori_loop(0, NPAIR, pair, 0)

    plsc.subcore_barrier()
    pltpu.sync_copy(agg_sh.at[pl.ds(rowo, SL), :],
                    agg_out.at[c, pl.ds(rowo, SL), :])
    pltpu.sync_copy(den_sh.at[pl.ds(rowo, SL)], den_out.at[c, pl.ds(rowo, SL)])


@functools.partial(
    pl.kernel, mesh=_mesh, compiler_params=_sc_params,
    out_type=(jax.ShapeDtypeStruct((E_PAD,), jnp.float32),
              jax.ShapeDtypeStruct((2, NPAD, D_EMB), jnp.float32),
              jax.ShapeDtypeStruct((2, NPAD), jnp.float32)),
    scratch_types=[
        pltpu.VMEM((BLK,), jnp.int32),
        pltpu.VMEM((BLK,), jnp.int32),
        pltpu.VMEM((BLK,), jnp.int32),
        pltpu.VMEM((BLK,), jnp.int32),
        pltpu.VMEM((BLK, D_EMB), jnp.float32),
        pltpu.VMEM((BLK, D_EMB), jnp.float32),
        pltpu.VMEM((BLK,), jnp.float32),
        pltpu.VMEM((BLK,), jnp.float32),
        pltpu.VMEM((NPAD,), jnp.float32),
        pltpu.VMEM((NPAD,), jnp.float32),
        pltpu.VMEM((SL // 2, D_EMB), jnp.float32),
        pltpu.VMEM((SL,), jnp.float32),
        pltpu.VMEM_SHARED((NPAD, D_EMB), jnp.float32),
        pltpu.VMEM_SHARED((NPAD,), jnp.float32),
        pltpu.SemaphoreType.DMA,
        pltpu.SemaphoreType.DMA,
    ])
def _k2(xl_hbm, src_hbm, dst_hbm, e_hbm, m_hbm, ee_out, agg_out, den_out,
        *scratch):
    _k2_body(xl_hbm, src_hbm, dst_hbm, e_hbm, m_hbm, ee_out, agg_out, den_out,
             *scratch)


def _k3_body(ee_hbm, dst_hbm, den_hbm, alpha_out, idx_b, ee_blk, den_loc):
    c = lax.axis_index("c")
    s = lax.axis_index("s")
    wid = s * 2 + c
    base = pl.multiple_of(wid * CHUNK, 8)
    for l in range(N_LAYERS):
        pltpu.sync_copy(den_hbm.at[l], den_loc)

        def blk_body(b, _):
            off = pl.multiple_of(base + b * BLK, 8)
            pltpu.sync_copy(dst_hbm.at[pl.ds(off, BLK)], idx_b)
            pltpu.sync_copy(ee_hbm.at[l, pl.ds(off, BLK)], ee_blk)

            def grp(g, _):
                sl = pl.ds(g * 16, 16)
                dn = plsc.load_gather(den_loc, [idx_b[sl]])
                ee_blk[sl] = ee_blk[sl] / (dn + 1e-16)
                return 0
            lax.fori_loop(0, GPB, grp, 0)
            pltpu.sync_copy(ee_blk, alpha_out.at[l, pl.ds(off, BLK)])
            return 0
        lax.fori_loop(0, NBLK, blk_body, 0)


@functools.partial(
    pl.kernel, mesh=_mesh, compiler_params=_sc_params,
    out_type=jax.ShapeDtypeStruct((N_LAYERS, E_PAD), jnp.float32),
    scratch_types=[
        pltpu.VMEM((BLK,), jnp.int32),
        pltpu.VMEM((BLK,), jnp.float32),
        pltpu.VMEM((NPAD,), jnp.float32),
    ])
def _k3(ee_hbm, dst_hbm, den_hbm, alpha_out, *scratch):
    _k3_body(ee_hbm, dst_hbm, den_hbm, alpha_out, *scratch)


def kernel(x, edge_index, gamma, beta, run_mean, run_var, W_in, b_in,
           W_l, b_l, att, conv_b, gn_w, gn_b, gn_ms, get_attention_weights):
    loops = jnp.arange(N_NODES, dtype=jnp.int32)
    padv = jnp.full((E_PAD - E_TOT,), N_NODES, jnp.int32)
    src = jnp.concatenate([edge_index[0].astype(jnp.int32), loops, padv])
    dst = jnp.concatenate([edge_index[1].astype(jnp.int32), loops, padv])

    xl = _tc_front(x, gamma, beta, run_mean, run_var, W_in, b_in,
                   W_l[0], b_l[0])
    ee_list, den_list = [], []
    h = None
    for l in range(N_LAYERS):
        e, m = _k1(xl, src, dst, att[l])
        ee, aggu, den = _k2(xl, src, dst, e, m)
        ee_list.append(ee)
        den2 = den.reshape(2, NPAD, 1)
        if l + 1 < N_LAYERS:
            xl, dcomb = _tc_mid(aggu, den2, conv_b[l], gn_w[l], gn_b[l],
                                gn_ms[l], W_l[l + 1], b_l[l + 1])
        else:
            h, dcomb = _tc_last(aggu, den2, conv_b[l], gn_w[l], gn_b[l],
                                gn_ms[l])
        den_list.append(dcomb)

    ee_all = jnp.stack(ee_list)
    den_all = jnp.stack([d.reshape(NPAD) for d in den_list])
    alpha = _k3(ee_all, dst, den_all)
    attns = alpha[:, :E_TOT]
    return (h, h, attns)


# speedup vs baseline: 1.0573x; 1.0573x over previous
import functools

import jax
import jax.numpy as jnp
from jax import lax
from jax.experimental import pallas as pl
from jax.experimental.pallas import tpu as pltpu
from jax.experimental.pallas import tpu_sc as plsc

N_NODES = 10000
D_EMB = 64
N_LAYERS = 4
N_EDGES_IN = 320000
E_TOT = N_EDGES_IN + N_NODES
NPAD = 10240
E_PAD = 331776
NW = 32
CHUNK = E_PAD // NW
BLK = 288
NBLK = CHUNK // BLK
NPAIR = NBLK // 2
GPB = BLK // 16
SL = NPAD // 16

_mesh = plsc.VectorSubcoreMesh(core_axis_name="c", subcore_axis_name="s")
_sc_params = pltpu.CompilerParams(needs_layout_passes=False,
                                  use_tc_tiling_on_sc=False)


def _leaky(v, s):
    return jnp.maximum(v, 0.0) + s * jnp.minimum(v, 0.0)


def _tc_front_body(x_ref, gm_ref, bt_ref, mu_ref, vr_ref, wi_ref, bi_ref,
                   w0_ref, b0_ref, xl_ref):
    x = x_ref[...]
    inv = lax.rsqrt(vr_ref[...] + 1e-5)
    h = (x - mu_ref[...]) * inv * gm_ref[...] + bt_ref[...]
    h = _leaky(jnp.dot(h, wi_ref[...], preferred_element_type=jnp.float32)
               + bi_ref[...], 0.01)
    xl = jnp.dot(h, w0_ref[...], preferred_element_type=jnp.float32) + b0_ref[...]
    xl_ref[...] = jnp.zeros((NPAD, D_EMB), jnp.float32)
    xl_ref[0:N_NODES, :] = xl


def _tc_front(x, gamma, beta, run_mean, run_var, W_in, b_in, W0, b0):
    return pl.pallas_call(
        _tc_front_body,
        out_shape=jax.ShapeDtypeStruct((NPAD, D_EMB), jnp.float32),
    )(x, gamma.reshape(1, -1), beta.reshape(1, -1), run_mean.reshape(1, -1),
      run_var.reshape(1, -1), W_in, b_in.reshape(1, -1), W0, b0.reshape(1, -1))


def _graphnorm(aggu_ref, den_ref, cb_ref, gw_ref, gb_ref, gms_ref):
    a = aggu_ref[0] + aggu_ref[1]
    den_all = den_ref[0] + den_ref[1]
    agg = (a[0:N_NODES, :] / (den_all[0:N_NODES, :] + 1e-16)) + cb_ref[...]
    mean = jnp.mean(agg, axis=0, keepdims=True)
    out = agg - mean * gms_ref[...]
    var = jnp.mean(out * out, axis=0, keepdims=True)
    h = gw_ref[...] * out * lax.rsqrt(var + 1e-5) + gb_ref[...]
    return _leaky(h, 0.01), den_all


def _tc_mid_body(aggu_ref, den_ref, cb_ref, gw_ref, gb_ref, gms_ref, w_ref,
                 b_ref, xl_ref, dout_ref):
    h, den_all = _graphnorm(aggu_ref, den_ref, cb_ref, gw_ref, gb_ref, gms_ref)
    xl = jnp.dot(h, w_ref[...], preferred_element_type=jnp.float32) + b_ref[...]
    xl_ref[...] = jnp.zeros((NPAD, D_EMB), jnp.float32)
    xl_ref[0:N_NODES, :] = xl
    dout_ref[...] = den_all


def _tc_mid(aggu, den, conv_b, gn_w, gn_b, gn_ms, W, b):
    return pl.pallas_call(
        _tc_mid_body,
        out_shape=(jax.ShapeDtypeStruct((NPAD, D_EMB), jnp.float32),
                   jax.ShapeDtypeStruct((NPAD, 1), jnp.float32)),
    )(aggu, den, conv_b.reshape(1, -1), gn_w.reshape(1, -1),
      gn_b.reshape(1, -1), gn_ms.reshape(1, -1), W, b.reshape(1, -1))


def _tc_last_body(aggu_ref, den_ref, cb_ref, gw_ref, gb_ref, gms_ref, h_ref,
                  dout_ref):
    h, den_all = _graphnorm(aggu_ref, den_ref, cb_ref, gw_ref, gb_ref, gms_ref)
    h_ref[...] = h
    dout_ref[...] = den_all


def _tc_last(aggu, den, conv_b, gn_w, gn_b, gn_ms):
    return pl.pallas_call(
        _tc_last_body,
        out_shape=(jax.ShapeDtypeStruct((N_NODES, D_EMB), jnp.float32),
                   jax.ShapeDtypeStruct((NPAD, 1), jnp.float32)),
    )(aggu, den, conv_b.reshape(1, -1), gn_w.reshape(1, -1),
      gn_b.reshape(1, -1), gn_ms.reshape(1, -1))


def _k1_body(xl_hbm, src_hbm, dst_hbm, att_hbm, e_out, m_out,
             src_l, dst_l, rows0, rows1, e_chunk, m_loc, att_v, m_ca, m_cb,
             m_sh, sem0, sem1):
    c = lax.axis_index("c")
    s = lax.axis_index("s")
    wid = s * 2 + c
    base = pl.multiple_of(wid * CHUNK, 8)
    pltpu.sync_copy(att_hbm, att_v)
    pltpu.sync_copy(src_hbm.at[pl.ds(base, CHUNK)], src_l)
    pltpu.sync_copy(dst_hbm.at[pl.ds(base, CHUNK)], dst_l)

    neg = jnp.full((16,), -3e38, jnp.float32)

    def init_b(i, _):
        m_loc[pl.ds(i * 16, 16)] = neg
        return 0
    lax.fori_loop(0, NPAD // 16, init_b, 0)

    att_c = [att_v[pl.ds(k * 16, 16)] for k in range(D_EMB // 16)]

    def issue(b, rows, sem):
        o = pl.multiple_of(b * BLK, 8)
        pltpu.async_copy(xl_hbm.at[src_l.at[pl.ds(o, BLK)]],
                         rows.at[pl.ds(0, BLK), :], sem)
        pltpu.async_copy(xl_hbm.at[dst_l.at[pl.ds(o, BLK)]],
                         rows.at[pl.ds(BLK, BLK), :], sem)

    def wait(b, rows, sem):
        o = pl.multiple_of(b * BLK, 8)
        pltpu.make_async_copy(xl_hbm.at[src_l.at[pl.ds(o, BLK)]],
                              rows.at[pl.ds(0, BLK), :], sem).wait()
        pltpu.make_async_copy(xl_hbm.at[dst_l.at[pl.ds(o, BLK)]],
                              rows.at[pl.ds(BLK, BLK), :], sem).wait()

    def compute(b, rows):
        def grp(g, _):
            ids = g * 16 + lax.iota(jnp.int32, 16)
            acc = jnp.zeros((16,), jnp.float32)
            for d in range(D_EMB):
                dd = jnp.full((16,), d, jnp.int32)
                v = (plsc.load_gather(rows, [ids, dd])
                     + plsc.load_gather(rows, [BLK + ids, dd]))
                acc = acc + att_c[d // 16][d % 16] * _leaky(v, 0.2)
            eo = b * BLK + g * 16
            e_chunk[pl.ds(eo, 16)] = acc

            dst_v = dst_l[pl.ds(eo, 16)]

            def upd_cond(pending):
                return jnp.any(pending)

            def upd_body(pending):
                cur = plsc.load_gather(m_loc, [dst_v])
                plsc.store_scatter(m_loc, [dst_v],
                                   jnp.maximum(cur, acc), mask=pending)
                chk = plsc.load_gather(m_loc, [dst_v])
                return pending & (chk < acc)
            lax.while_loop(upd_cond, upd_body, jnp.ones((16,), jnp.bool_))
            return 0
        lax.fori_loop(0, GPB, grp, 0)

    issue(0, rows0, sem0)

    def pair(h, _):
        b0 = 2 * h
        issue(b0 + 1, rows1, sem1)
        wait(b0, rows0, sem0)
        compute(b0, rows0)

        @pl.when(h + 1 < NPAIR)
        def _():
            issue(b0 + 2, rows0, sem0)
        wait(b0 + 1, rows1, sem1)
        compute(b0 + 1, rows1)
        return 0
    lax.fori_loop(0, NPAIR, pair, 0)
    pltpu.sync_copy(e_chunk, e_out.at[pl.ds(base, CHUNK)])

    pltpu.sync_copy(m_loc, m_sh.at[s])
    plsc.subcore_barrier()
    colo = pl.multiple_of(s * SL, 8)
    pltpu.sync_copy(m_sh.at[0, pl.ds(colo, SL)], m_ca)

    def red_j(j, _):
        pltpu.sync_copy(m_sh.at[j, pl.ds(colo, SL)], m_cb)

        def mx(k, _):
            sl = pl.ds(k * 16, 16)
            m_ca[sl] = jnp.maximum(m_ca[sl], m_cb[sl])
            return 0
        lax.fori_loop(0, SL // 16, mx, 0)
        return 0
    lax.fori_loop(1, 16, red_j, 0)
    pltpu.sync_copy(m_ca, m_out.at[c, pl.ds(colo, SL)])


@functools.partial(
    pl.kernel, mesh=_mesh, compiler_params=_sc_params,
    out_type=(jax.ShapeDtypeStruct((E_PAD,), jnp.float32),
              jax.ShapeDtypeStruct((2, NPAD), jnp.float32)),
    scratch_types=[
        pltpu.VMEM((CHUNK,), jnp.int32),
        pltpu.VMEM((CHUNK,), jnp.int32),
        pltpu.VMEM((2 * BLK, D_EMB), jnp.float32),
        pltpu.VMEM((2 * BLK, D_EMB), jnp.float32),
        pltpu.VMEM((CHUNK,), jnp.float32),
        pltpu.VMEM((NPAD,), jnp.float32),
        pltpu.VMEM((D_EMB,), jnp.float32),
        pltpu.VMEM((SL,), jnp.float32),
        pltpu.VMEM((SL,), jnp.float32),
        pltpu.VMEM_SHARED((16, NPAD), jnp.float32),
        pltpu.SemaphoreType.DMA,
        pltpu.SemaphoreType.DMA,
    ])
def _k1(xl_hbm, src_hbm, dst_hbm, att_hbm, e_out, m_out, *scratch):
    _k1_body(xl_hbm, src_hbm, dst_hbm, att_hbm, e_out, m_out, *scratch)


def _k2_body(xl_hbm, src_hbm, dst_hbm, e_hbm, m_hbm, ee_out, agg_out, den_out,
             si0, si1, di0, di1, rows0, rows1, e_chunk, m_loc, m_tmp,
             zbuf, d_ca, agg_sh, den_sh, semg0, semg1):
    c = lax.axis_index("c")
    s = lax.axis_index("s")
    wid = s * 2 + c
    base = pl.multiple_of(wid * CHUNK, 8)
    pltpu.sync_copy(e_hbm.at[pl.ds(base, CHUNK)], e_chunk)

    pltpu.sync_copy(m_hbm.at[0], m_loc)
    pltpu.sync_copy(m_hbm.at[1], m_tmp)

    def mx(k, _):
        sl = pl.ds(k * 16, 16)
        m_loc[sl] = jnp.maximum(m_loc[sl], m_tmp[sl])
        return 0
    lax.fori_loop(0, NPAD // 16, mx, 0)

    zero16 = jnp.zeros((16,), jnp.float32)

    def zr(r, _):
        def zc(k, _):
            zbuf[r, pl.ds(k * 16, 16)] = zero16
            return 0
        lax.fori_loop(0, D_EMB // 16, zc, 0)
        return 0
    lax.fori_loop(0, SL // 4, zr, 0)

    def zv(k, _):
        d_ca[pl.ds(k * 16, 16)] = zero16
        return 0
    lax.fori_loop(0, SL // 16, zv, 0)
    rowo = pl.multiple_of(s * SL, 8)
    colo = pl.multiple_of(s * SL, 8)
    for q in range(4):
        pltpu.sync_copy(zbuf, agg_sh.at[pl.ds(rowo + q * (SL // 4), SL // 4), :])
    pltpu.sync_copy(d_ca, den_sh.at[pl.ds(colo, SL)])
    plsc.subcore_barrier()

    def issue(b, rows, si, di, semg):
        o = pl.multiple_of(b * BLK, 8)
        pltpu.sync_copy(src_hbm.at[pl.ds(base + o, BLK)], si)
        pltpu.async_copy(dst_hbm.at[pl.ds(base + o, BLK)], di, semg)
        pltpu.async_copy(xl_hbm.at[si], rows, semg)

    def waitg(b, rows, si, di, semg):
        o = pl.multiple_of(b * BLK, 8)
        pltpu.make_async_copy(dst_hbm.at[pl.ds(base + o, BLK)],
                              di, semg).wait()
        pltpu.make_async_copy(xl_hbm.at[si], rows, semg).wait()

    def scat(b, rows, di):
        o = pl.multiple_of(b * BLK, 8)
        pltpu.sync_copy(rows, agg_sh.at[di], add=True)
        pltpu.sync_copy(e_chunk.at[pl.ds(o, BLK)], den_sh.at[di], add=True)

    def compute(b, rows, di):
        def grp(g, _):
            eo = b * BLK + g * 16
            sl = pl.ds(eo, 16)
            ids = g * 16 + lax.iota(jnp.int32, 16)
            dst_v = di[pl.ds(g * 16, 16)]
            mg = plsc.load_gather(m_loc, [dst_v])
            ee = jnp.exp(e_chunk[sl] - mg)
            e_chunk[sl] = ee

            for d in range(D_EMB):
                dd = jnp.full((16,), d, jnp.int32)
                col = plsc.load_gather(rows, [ids, dd])
                plsc.store_scatter(rows, [ids, dd], col * ee)
            return 0
        lax.fori_loop(0, GPB, grp, 0)

    issue(0, rows0, si0, di0, semg0)

    def pair(h, _):
        b0 = 2 * h
        issue(b0 + 1, rows1, si1, di1, semg1)
        waitg(b0, rows0, si0, di0, semg0)
        compute(b0, rows0, di0)
        scat(b0, rows0, di0)

        @pl.when(h + 1 < NPAIR)
        def _():
            issue(b0 + 2, rows0, si0, di0, semg0)
        waitg(b0 + 1, rows1, si1, di1, semg1)
        compute(b0 + 1, rows1, di1)
        scat(b0 + 1, rows1, di1)
        return 0
    lax.fori_loop(0, NPAIR, pair, 0)
    pltpu.sync_copy(e_chunk, ee_out.at[pl.ds(base, CHUNK)])

    plsc.subcore_barrier()
    pltpu.sync_copy(agg_sh.at[pl.ds(rowo, SL), :],
                    agg_out.at[c, pl.ds(rowo, SL), :])
    pltpu.sync_copy(den_sh.at[pl.ds(colo, SL)], den_out.at[c, pl.ds(colo, SL)])


@functools.partial(
    pl.kernel, mesh=_mesh, compiler_params=_sc_params,
    out_type=(jax.ShapeDtypeStruct((E_PAD,), jnp.float32),
              jax.ShapeDtypeStruct((2, NPAD, D_EMB), jnp.float32),
              jax.ShapeDtypeStruct((2, NPAD), jnp.float32)),
    scratch_types=[
        pltpu.VMEM((BLK,), jnp.int32),
        pltpu.VMEM((BLK,), jnp.int32),
        pltpu.VMEM((BLK,), jnp.int32),
        pltpu.VMEM((BLK,), jnp.int32),
        pltpu.VMEM((BLK, D_EMB), jnp.float32),
        pltpu.VMEM((BLK, D_EMB), jnp.float32),
        pltpu.VMEM((CHUNK,), jnp.float32),
        pltpu.VMEM((NPAD,), jnp.float32),
        pltpu.VMEM((NPAD,), jnp.float32),
        pltpu.VMEM((SL // 4, D_EMB), jnp.float32),
        pltpu.VMEM((SL,), jnp.float32),
        pltpu.VMEM_SHARED((NPAD, D_EMB), jnp.float32),
        pltpu.VMEM_SHARED((NPAD,), jnp.float32),
        pltpu.SemaphoreType.DMA,
        pltpu.SemaphoreType.DMA,
    ])
def _k2(xl_hbm, src_hbm, dst_hbm, e_hbm, m_hbm, ee_out, agg_out, den_out,
        *scratch):
    _k2_body(xl_hbm, src_hbm, dst_hbm, e_hbm, m_hbm, ee_out, agg_out, den_out,
             *scratch)


def _k3_body(ee_hbm, dst_hbm, den_hbm, alpha_out, dst_l, e_chunk, den_loc):
    c = lax.axis_index("c")
    s = lax.axis_index("s")
    wid = s * 2 + c
    base = pl.multiple_of(wid * CHUNK, 8)
    pltpu.sync_copy(dst_hbm.at[pl.ds(base, CHUNK)], dst_l)
    for l in range(N_LAYERS):
        pltpu.sync_copy(den_hbm.at[l], den_loc)
        pltpu.sync_copy(ee_hbm.at[l, pl.ds(base, CHUNK)], e_chunk)

        def grp(g, _):
            sl = pl.ds(g * 16, 16)
            dn = plsc.load_gather(den_loc, [dst_l[sl]])
            e_chunk[sl] = e_chunk[sl] / (dn + 1e-16)
            return 0
        lax.fori_loop(0, CHUNK // 16, grp, 0)
        pltpu.sync_copy(e_chunk, alpha_out.at[l, pl.ds(base, CHUNK)])


@functools.partial(
    pl.kernel, mesh=_mesh, compiler_params=_sc_params,
    out_type=jax.ShapeDtypeStruct((N_LAYERS, E_PAD), jnp.float32),
    scratch_types=[
        pltpu.VMEM((CHUNK,), jnp.int32),
        pltpu.VMEM((CHUNK,), jnp.float32),
        pltpu.VMEM((NPAD,), jnp.float32),
    ])
def _k3(ee_hbm, dst_hbm, den_hbm, alpha_out, *scratch):
    _k3_body(ee_hbm, dst_hbm, den_hbm, alpha_out, *scratch)


def kernel(x, edge_index, gamma, beta, run_mean, run_var, W_in, b_in,
           W_l, b_l, att, conv_b, gn_w, gn_b, gn_ms, get_attention_weights):
    loops = jnp.arange(N_NODES, dtype=jnp.int32)
    padv = jnp.full((E_PAD - E_TOT,), N_NODES, jnp.int32)
    src = jnp.concatenate([edge_index[0].astype(jnp.int32), loops, padv])
    dst = jnp.concatenate([edge_index[1].astype(jnp.int32), loops, padv])

    xl = _tc_front(x, gamma, beta, run_mean, run_var, W_in, b_in,
                   W_l[0], b_l[0])
    ee_list, den_list = [], []
    h = None
    for l in range(N_LAYERS):
        e, m = _k1(xl, src, dst, att[l])
        ee, aggu, den = _k2(xl, src, dst, e, m)
        ee_list.append(ee)
        den2 = den.reshape(2, NPAD, 1)
        if l + 1 < N_LAYERS:
            xl, dcomb = _tc_mid(aggu, den2, conv_b[l], gn_w[l], gn_b[l],
                                gn_ms[l], W_l[l + 1], b_l[l + 1])
        else:
            h, dcomb = _tc_last(aggu, den2, conv_b[l], gn_w[l], gn_b[l],
                                gn_ms[l])
        den_list.append(dcomb)

    ee_all = jnp.stack(ee_list)
    den_all = jnp.stack([d.reshape(NPAD) for d in den_list])
    alpha = _k3(ee_all, dst, den_all)
    attns = alpha[:, :E_TOT]
    return (h, h, attns)
